# baseline (speedup 1.0000x reference)
import functools
import math

import jax
import jax.numpy as jnp
from jax.experimental import pallas as pl
from jax.experimental.pallas import tpu as pltpu


def _round_up(x, m):
    return ((x + m - 1) // m) * m


def _ff_kernel(x_ref, w_ref, o_ref, *, enc_dim):
    proj = jnp.dot(x_ref[...], w_ref[...], preferred_element_type=jnp.float32)
    lane = jax.lax.broadcasted_iota(jnp.int32, (1, proj.shape[-1]), 1)
    phase = jnp.where(lane >= enc_dim, jnp.float32(jnp.pi / 2), jnp.float32(0.0))
    o_ref[...] = jnp.sin(proj + phase).astype(o_ref.dtype)


def kernel(pos, B):
    pos_dim, enc_dim = B.shape
    f_dim = 2 * enc_dim
    assert pos.shape[-1] == pos_dim
    lead = pos.shape[:-1]
    M = math.prod(lead) if lead else 1
    out_dtype = pos.dtype

    x2d = pos.reshape(M, pos_dim)
    w = jnp.concatenate([B, B], axis=1).astype(pos.dtype)

    tbr = max(8, min(8192, _round_up(M, 8)))
    n_tiles = pl.cdiv(M, tbr)
    m_pad = n_tiles * tbr
    if m_pad > M:
        x2d = jnp.pad(x2d, ((0, m_pad - M), (0, 0)))

    out2d = pl.pallas_call(
        functools.partial(_ff_kernel, enc_dim=enc_dim),
        out_shape=jax.ShapeDtypeStruct((m_pad, f_dim), out_dtype),
        grid=(n_tiles,),
        in_specs=[
            pl.BlockSpec((tbr, pos_dim), lambda i: (i, 0)),
            pl.BlockSpec((pos_dim, f_dim), lambda i: (0, 0)),
        ],
        out_specs=pl.BlockSpec((tbr, f_dim), lambda i: (i, 0)),
        compiler_params=pltpu.CompilerParams(
            dimension_semantics=("parallel",),
            vmem_limit_bytes=64 * 1024 * 1024,
        ),
    )(x2d, w)
    if m_pad > M:
        out2d = out2d[:M]
    return out2d.reshape(*lead, f_dim)


# speedup vs baseline: 1.2622x; 1.2622x over previous
import functools
import math

import jax
import jax.numpy as jnp
from jax.experimental import pallas as pl
from jax.experimental.pallas import tpu as pltpu


def _ff3d_kernel(x_ref, w_ref, o_ref, *, enc_dim):
    w = w_ref[...]
    f_dim = 2 * enc_dim
    lane = jax.lax.broadcasted_iota(jnp.int32, (1, f_dim), 1)
    phase = jnp.where(lane >= enc_dim, jnp.float32(jnp.pi / 2), jnp.float32(0.0))
    for t in range(x_ref.shape[0]):
        proj = jnp.dot(x_ref[t], w, preferred_element_type=jnp.float32)
        o_ref[t] = jnp.sin(proj + phase).astype(o_ref.dtype)


def kernel(pos, B):
    pos_dim, enc_dim = B.shape
    f_dim = 2 * enc_dim
    assert pos.shape[-1] == pos_dim and pos.ndim == 3
    n, s = pos.shape[0], pos.shape[1]
    out_dtype = pos.dtype

    w = jnp.concatenate([B, B], axis=1).astype(pos.dtype)

    tb = 16
    while n % tb:
        tb //= 2
    n_tiles = n // tb

    return pl.pallas_call(
        functools.partial(_ff3d_kernel, enc_dim=enc_dim),
        out_shape=jax.ShapeDtypeStruct((n, s, f_dim), out_dtype),
        grid=(n_tiles,),
        in_specs=[
            pl.BlockSpec((tb, s, pos_dim), lambda i: (i, 0, 0)),
            pl.BlockSpec((pos_dim, f_dim), lambda i: (0, 0)),
        ],
        out_specs=pl.BlockSpec((tb, s, f_dim), lambda i: (i, 0, 0)),
        compiler_params=pltpu.CompilerParams(
            dimension_semantics=("parallel",),
            vmem_limit_bytes=64 * 1024 * 1024,
        ),
    )(pos, w)


# speedup vs baseline: 2.8868x; 2.2870x over previous
import functools
import math

import jax
import jax.numpy as jnp
from jax.experimental import pallas as pl
from jax.experimental.pallas import tpu as pltpu

_INV_PI = 0.31830987334251404
_MAGIC = 12582912.0
_PI1 = 3.140625
_PI2 = 0.0009676535846665502
_PI3 = 5.126565838509123e-12
_C3 = -1.0 / 6.0
_C5 = 1.0 / 120.0
_C7 = -1.0 / 5040.0
_C9 = 1.0 / 362880.0


def _fast_sin(x):
    f32 = jnp.float32
    t = x * f32(_INV_PI) + f32(_MAGIC)
    ti = jax.lax.bitcast_convert_type(t, jnp.int32)
    sign = jax.lax.shift_left(jax.lax.bitwise_and(ti, 1), 31)
    kf = t - f32(_MAGIC)
    r = x - kf * f32(_PI1)
    r = r - kf * f32(_PI2)
    r = r - kf * f32(_PI3)
    r2 = r * r
    p = r * (f32(1.0) + r2 * (f32(_C3) + r2 * (f32(_C5) + r2 * (f32(_C7) + r2 * f32(_C9)))))
    pbits = jax.lax.bitcast_convert_type(p, jnp.int32)
    return jax.lax.bitcast_convert_type(jax.lax.bitwise_xor(pbits, sign), jnp.float32)


def _ff3d_kernel(x_ref, w_ref, o_ref, *, enc_dim):
    w = w_ref[...]
    f_dim = 2 * enc_dim
    lane = jax.lax.broadcasted_iota(jnp.int32, (1, f_dim), 1)
    phase = jnp.where(lane >= enc_dim, jnp.float32(jnp.pi / 2), jnp.float32(0.0))
    for t in range(x_ref.shape[0]):
        proj = jnp.dot(x_ref[t], w, preferred_element_type=jnp.float32)
        o_ref[t] = _fast_sin(proj + phase).astype(o_ref.dtype)


def kernel(pos, B):
    pos_dim, enc_dim = B.shape
    f_dim = 2 * enc_dim
    assert pos.shape[-1] == pos_dim and pos.ndim == 3
    n, s = pos.shape[0], pos.shape[1]
    out_dtype = pos.dtype

    w = jnp.concatenate([B, B], axis=1).astype(pos.dtype)

    tb = 16
    while n % tb:
        tb //= 2
    n_tiles = n // tb

    return pl.pallas_call(
        functools.partial(_ff3d_kernel, enc_dim=enc_dim),
        out_shape=jax.ShapeDtypeStruct((n, s, f_dim), out_dtype),
        grid=(n_tiles,),
        in_specs=[
            pl.BlockSpec((tb, s, pos_dim), lambda i: (i, 0, 0)),
            pl.BlockSpec((pos_dim, f_dim), lambda i: (0, 0)),
        ],
        out_specs=pl.BlockSpec((tb, s, f_dim), lambda i: (i, 0, 0)),
        compiler_params=pltpu.CompilerParams(
            dimension_semantics=("parallel",),
            vmem_limit_bytes=64 * 1024 * 1024,
        ),
    )(pos, w)


# speedup vs baseline: 25.5227x; 8.8411x over previous
import functools
import math

import jax
import jax.numpy as jnp
from jax.experimental import pallas as pl
from jax.experimental.pallas import tpu as pltpu

_INV_PI = 0.31830987334251404
_MAGIC = 12582912.0
_PI1 = 3.140625
_PI2 = 0.0009676535846665502
_PI3 = 5.126565838509123e-12
_C3 = -1.0 / 6.0
_C5 = 1.0 / 120.0
_C7 = -1.0 / 5040.0
_C9 = 1.0 / 362880.0


def _fast_sin(x):
    f32 = jnp.float32
    t = x * f32(_INV_PI) + f32(_MAGIC)
    ti = jax.lax.bitcast_convert_type(t, jnp.int32)
    sign = jax.lax.shift_left(jax.lax.bitwise_and(ti, 1), 31)
    kf = t - f32(_MAGIC)
    r = x - kf * f32(_PI1)
    r = r - kf * f32(_PI2)
    r = r - kf * f32(_PI3)
    r2 = r * r
    p = r * (f32(1.0) + r2 * (f32(_C3) + r2 * (f32(_C5) + r2 * (f32(_C7) + r2 * f32(_C9)))))
    pbits = jax.lax.bitcast_convert_type(p, jnp.int32)
    return jax.lax.bitcast_convert_type(jax.lax.bitwise_xor(pbits, sign), jnp.float32)


def _ffT_kernel(x_ref, wt_ref, o_ref, *, enc_dim):
    wt = wt_ref[...]
    f_dim = 2 * enc_dim
    feat = jax.lax.broadcasted_iota(jnp.int32, (f_dim, 1), 0)
    phase = jnp.where(feat >= enc_dim, jnp.float32(jnp.pi / 2), jnp.float32(0.0))
    for t in range(x_ref.shape[0]):
        proj = jnp.dot(wt, x_ref[t], preferred_element_type=jnp.float32)
        o_ref[t] = _fast_sin(proj + phase).astype(o_ref.dtype)


def kernel(pos, B):
    pos_dim, enc_dim = B.shape
    f_dim = 2 * enc_dim
    assert pos.shape[-1] == pos_dim and pos.ndim == 3
    n, s = pos.shape[0], pos.shape[1]
    out_dtype = pos.dtype

    x_t = jnp.transpose(pos, (0, 2, 1))
    wt = jnp.concatenate([B, B], axis=1).T.astype(pos.dtype)

    tb = 64
    while n % tb:
        tb //= 2
    n_tiles = n // tb

    out_t = pl.pallas_call(
        functools.partial(_ffT_kernel, enc_dim=enc_dim),
        out_shape=jax.ShapeDtypeStruct((n, f_dim, s), out_dtype),
        grid=(n_tiles,),
        in_specs=[
            pl.BlockSpec((tb, pos_dim, s), lambda i: (i, 0, 0)),
            pl.BlockSpec((f_dim, pos_dim), lambda i: (0, 0)),
        ],
        out_specs=pl.BlockSpec((tb, f_dim, s), lambda i: (i, 0, 0)),
        compiler_params=pltpu.CompilerParams(
            dimension_semantics=("parallel",),
            vmem_limit_bytes=100 * 1024 * 1024,
        ),
    )(x_t, wt)
    return jnp.transpose(out_t, (0, 2, 1))


# speedup vs baseline: 37.7398x; 1.4787x over previous
import functools
import math

import jax
import jax.numpy as jnp
from jax.experimental import pallas as pl
from jax.experimental.pallas import tpu as pltpu

_INV_PI = 0.31830987334251404
_MAGIC = 12582912.0
_PI1 = 3.140625
_PI2 = 0.0009676535846665502
_PI3 = 5.126565838509123e-12
_S3 = -1.0 / 6.0
_S5 = 1.0 / 120.0
_S7 = -1.0 / 5040.0
_S9 = 1.0 / 362880.0
_C2 = -0.5
_C4 = 1.0 / 24.0
_C6 = -1.0 / 720.0
_C8 = 1.0 / 40320.0


def _ffT_kernel(x_ref, wt_ref, o_ref, *, enc_dim):
    f32 = jnp.float32
    wt = wt_ref[...]
    for t in range(x_ref.shape[0]):
        proj = jnp.dot(wt, x_ref[t], preferred_element_type=jnp.float32)
        tt = proj * f32(_INV_PI) + f32(_MAGIC)
        sign = jax.lax.shift_left(
            jax.lax.bitwise_and(jax.lax.bitcast_convert_type(tt, jnp.int32), 1), 31)
        kf = tt - f32(_MAGIC)
        r = proj - kf * f32(_PI1)
        r = r - kf * f32(_PI2)
        r = r - kf * f32(_PI3)
        r2 = r * r
        sinp = r * (f32(1.0) + r2 * (f32(_S3) + r2 * (f32(_S5) + r2 * (f32(_S7) + r2 * f32(_S9)))))
        cosp = f32(1.0) + r2 * (f32(_C2) + r2 * (f32(_C4) + r2 * (f32(_C6) + r2 * f32(_C8))))
        o_ref[t, :enc_dim] = jax.lax.bitcast_convert_type(
            jax.lax.bitwise_xor(jax.lax.bitcast_convert_type(sinp, jnp.int32), sign),
            jnp.float32).astype(o_ref.dtype)
        o_ref[t, enc_dim:] = jax.lax.bitcast_convert_type(
            jax.lax.bitwise_xor(jax.lax.bitcast_convert_type(cosp, jnp.int32), sign),
            jnp.float32).astype(o_ref.dtype)


def kernel(pos, B):
    pos_dim, enc_dim = B.shape
    f_dim = 2 * enc_dim
    assert pos.shape[-1] == pos_dim and pos.ndim == 3
    n, s = pos.shape[0], pos.shape[1]
    out_dtype = pos.dtype

    x_t = jnp.transpose(pos, (0, 2, 1))
    wt = B.T.astype(pos.dtype)

    tb = 64
    while n % tb:
        tb //= 2
    n_tiles = n // tb

    out_t = pl.pallas_call(
        functools.partial(_ffT_kernel, enc_dim=enc_dim),
        out_shape=jax.ShapeDtypeStruct((n, f_dim, s), out_dtype),
        grid=(n_tiles,),
        in_specs=[
            pl.BlockSpec((tb, pos_dim, s), lambda i: (i, 0, 0)),
            pl.BlockSpec((enc_dim, pos_dim), lambda i: (0, 0)),
        ],
        out_specs=pl.BlockSpec((tb, f_dim, s), lambda i: (i, 0, 0)),
        compiler_params=pltpu.CompilerParams(
            dimension_semantics=("parallel",),
            vmem_limit_bytes=100 * 1024 * 1024,
        ),
    )(x_t, wt)
    return jnp.transpose(out_t, (0, 2, 1))


# speedup vs baseline: 42.5751x; 1.1281x over previous
import functools
import math

import jax
import jax.numpy as jnp
from jax.experimental import pallas as pl
from jax.experimental.pallas import tpu as pltpu

_INV_PI = 0.31830987334251404
_MAGIC = 12582912.0
_PI1 = 3.140625
_PI2 = 0.0009676535846665502
_S3 = -0.1666584
_S5 = 0.00831458
_S7 = -0.00018561
_C2 = -0.49994812
_C4 = 0.04152708
_C6 = -0.00128256


def _ffT_kernel(x_ref, wt_ref, o_ref, *, enc_dim):
    f32 = jnp.float32
    wt = wt_ref[...]
    for t in range(x_ref.shape[0]):
        proj = jnp.dot(wt, x_ref[t], preferred_element_type=jnp.float32)
        tt = proj * f32(_INV_PI) + f32(_MAGIC)
        sign = jax.lax.shift_left(
            jax.lax.bitwise_and(jax.lax.bitcast_convert_type(tt, jnp.int32), 1), 31)
        kf = tt - f32(_MAGIC)
        r = proj - kf * f32(_PI1)
        r = r - kf * f32(_PI2)
        r2 = r * r
        sinp = r * (f32(1.0) + r2 * (f32(_S3) + r2 * (f32(_S5) + r2 * f32(_S7))))
        cosp = f32(1.0) + r2 * (f32(_C2) + r2 * (f32(_C4) + r2 * f32(_C6)))
        o_ref[t, :enc_dim] = jax.lax.bitcast_convert_type(
            jax.lax.bitwise_xor(jax.lax.bitcast_convert_type(sinp, jnp.int32), sign),
            jnp.float32).astype(o_ref.dtype)
        o_ref[t, enc_dim:] = jax.lax.bitcast_convert_type(
            jax.lax.bitwise_xor(jax.lax.bitcast_convert_type(cosp, jnp.int32), sign),
            jnp.float32).astype(o_ref.dtype)


def kernel(pos, B):
    pos_dim, enc_dim = B.shape
    f_dim = 2 * enc_dim
    assert pos.shape[-1] == pos_dim and pos.ndim == 3
    n, s = pos.shape[0], pos.shape[1]
    out_dtype = pos.dtype

    x_t = jnp.transpose(pos, (0, 2, 1))
    wt = B.T.astype(pos.dtype)

    tb = 64
    while n % tb:
        tb //= 2
    n_tiles = n // tb

    out_t = pl.pallas_call(
        functools.partial(_ffT_kernel, enc_dim=enc_dim),
        out_shape=jax.ShapeDtypeStruct((n, f_dim, s), out_dtype),
        grid=(n_tiles,),
        in_specs=[
            pl.BlockSpec((tb, pos_dim, s), lambda i: (i, 0, 0)),
            pl.BlockSpec((enc_dim, pos_dim), lambda i: (0, 0)),
        ],
        out_specs=pl.BlockSpec((tb, f_dim, s), lambda i: (i, 0, 0)),
        compiler_params=pltpu.CompilerParams(
            dimension_semantics=("parallel",),
            vmem_limit_bytes=100 * 1024 * 1024,
        ),
    )(x_t, wt)
    return jnp.transpose(out_t, (0, 2, 1))
